# Initial kernel scaffold; baseline (speedup 1.0000x reference)
import jax
import jax.numpy as jnp
from jax.experimental import pallas as pl


def kernel(node_hidden, in_node_index, in_node_mask):
    raise NotImplementedError("write your pallas kernel here")


# speedup vs baseline: 57.3155x; 57.3155x over previous
import functools
import numpy as np
import jax
import jax.numpy as jnp
from jax import lax
from jax.experimental import pallas as pl
from jax.experimental.pallas import tpu as pltpu
from jax.experimental.pallas import tpu_sc as plsc

B, N, K, D = 16, 512, 32, 128
C = 128
NCHUNK = (B * N) // C
NW = 32
CHUNKS_PER_W = NCHUNK // NW
PAIRS = C * K
LANES = 16
GROUPS = PAIRS // LANES
RBLK = C // LANES


def _sc_scatter_body(cols_hbm, vals_hbm, rowpat_hbm, m_hbm,
                     idx_v, val_v, row_v, acc_v):
    wid = lax.axis_index("s") * 2 + lax.axis_index("c")
    pltpu.sync_copy(rowpat_hbm, row_v)
    for cc in range(CHUNKS_PER_W):
        chunk = wid * CHUNKS_PER_W + cc
        base = chunk * PAIRS
        pltpu.sync_copy(cols_hbm.at[pl.ds(base, PAIRS)], idx_v)
        pltpu.sync_copy(vals_hbm.at[pl.ds(base, PAIRS)], val_v)

        zeros = jnp.zeros((LANES,), jnp.float32)

        def zero_blk(i, carry):
            base0 = i * (LANES * 16)
            for j in range(16):
                acc_v[pl.ds(base0 + j * LANES, LANES)] = zeros
            return carry

        lax.fori_loop(0, (C * N) // (LANES * 16), zero_blk, 0)

        def group(g, carry):
            off = g * LANES
            rowb = row_v[pl.ds(off, LANES)]
            cols = idx_v[pl.ds(off, LANES)]
            vals = val_v[pl.ds(off, LANES)]
            plsc.addupdate_scatter(acc_v, [rowb + cols], vals)
            return carry

        lax.fori_loop(0, GROUPS, group, 0)
        pltpu.sync_copy(acc_v, m_hbm.at[chunk])


def _build_m_sc(cols_flat, vals_flat, rowpat):
    mesh = plsc.VectorSubcoreMesh(core_axis_name="c", subcore_axis_name="s",
                                  num_cores=2, num_subcores=16)
    k = pl.kernel(
        _sc_scatter_body,
        out_type=jax.ShapeDtypeStruct((NCHUNK, C * N), jnp.float32),
        mesh=mesh,
        scratch_types=[
            pltpu.VMEM((PAIRS,), jnp.int32),
            pltpu.VMEM((PAIRS,), jnp.float32),
            pltpu.VMEM((PAIRS,), jnp.int32),
            pltpu.VMEM((C * N,), jnp.float32),
        ],
        compiler_params=pltpu.CompilerParams(
            needs_layout_passes=False, use_tc_tiling_on_sc=False),
    )
    return k(cols_flat, vals_flat, rowpat)


def _mm_body(m_ref, h_ref, o_ref):
    m = m_ref[0]
    h1 = jnp.dot(m, h_ref[0], preferred_element_type=jnp.float32,
                 precision=lax.Precision.HIGHEST)
    o_ref[0] = jnp.dot(m, h1, preferred_element_type=jnp.float32,
                       precision=lax.Precision.HIGHEST)


def _two_layer_mm(m, h):
    return pl.pallas_call(
        _mm_body,
        grid=(B,),
        in_specs=[
            pl.BlockSpec((1, N, N), lambda b: (b, 0, 0)),
            pl.BlockSpec((1, N, D), lambda b: (b, 0, 0)),
        ],
        out_specs=pl.BlockSpec((1, N, D), lambda b: (b, 0, 0)),
        out_shape=jax.ShapeDtypeStruct((B, N, D), jnp.float32),
    )(m, h)


@jax.jit
def kernel(node_hidden, in_node_index, in_node_mask):
    idx4 = in_node_index.reshape(NCHUNK, C // LANES, LANES, K)
    cols_flat = idx4.transpose(0, 1, 3, 2).reshape(-1)
    msk4 = in_node_mask.reshape(NCHUNK, C // LANES, LANES, K)
    vals_flat = msk4.transpose(0, 1, 3, 2).reshape(-1)

    g = np.arange(PAIRS)
    rowpat = jnp.asarray(((g // (K * LANES)) * LANES + g % LANES) * N,
                         dtype=jnp.int32)

    m = _build_m_sc(cols_flat, vals_flat, rowpat)
    m = m.reshape(B, N, N)
    return _two_layer_mm(m, node_hidden)


# speedup vs baseline: 78.9087x; 1.3767x over previous
import functools
import numpy as np
import jax
import jax.numpy as jnp
from jax import lax
from jax.experimental import pallas as pl
from jax.experimental.pallas import tpu as pltpu
from jax.experimental.pallas import tpu_sc as plsc

B, N, K, D = 16, 512, 32, 128
C = 128
NCHUNK = (B * N) // C
NW = 32
CHUNKS_PER_W = NCHUNK // NW
PAIRS = C * K
LANES = 16
GROUPS = PAIRS // LANES
RBLK = C // LANES


def _sc_scatter_body(cols_hbm, vals_hbm, rowpat_hbm, m_hbm,
                     idx_v, val_v, row_v, acc_v):
    wid = lax.axis_index("s") * 2 + lax.axis_index("c")
    pltpu.sync_copy(rowpat_hbm, row_v)
    for cc in range(CHUNKS_PER_W):
        chunk = wid * CHUNKS_PER_W + cc
        base = chunk * PAIRS
        pltpu.sync_copy(cols_hbm.at[pl.ds(base, PAIRS)], idx_v)
        pltpu.sync_copy(vals_hbm.at[pl.ds(base, PAIRS)], val_v)

        zeros = jnp.zeros((LANES,), jnp.float32)

        def zero_blk(i, carry):
            base0 = i * (LANES * 16)
            for j in range(16):
                acc_v[pl.ds(base0 + j * LANES, LANES)] = zeros
            return carry

        lax.fori_loop(0, (C * N) // (LANES * 16), zero_blk, 0)

        def group(g, carry):
            off = g * LANES
            rowb = row_v[pl.ds(off, LANES)]
            cols = idx_v[pl.ds(off, LANES)]
            vals = val_v[pl.ds(off, LANES)]
            plsc.addupdate_scatter(acc_v, [rowb + cols], vals)
            return carry

        lax.fori_loop(0, GROUPS, group, 0)
        pltpu.sync_copy(acc_v, m_hbm.at[chunk])


def _build_m_sc(cols_flat, vals_flat, rowpat):
    mesh = plsc.VectorSubcoreMesh(core_axis_name="c", subcore_axis_name="s",
                                  num_cores=2, num_subcores=16)
    k = pl.kernel(
        _sc_scatter_body,
        out_type=jax.ShapeDtypeStruct((NCHUNK, C * N), jnp.float32),
        mesh=mesh,
        scratch_types=[
            pltpu.VMEM((PAIRS,), jnp.int32),
            pltpu.VMEM((PAIRS,), jnp.float32),
            pltpu.VMEM((PAIRS,), jnp.int32),
            pltpu.VMEM((C * N,), jnp.float32),
        ],
        compiler_params=pltpu.CompilerParams(
            needs_layout_passes=False, use_tc_tiling_on_sc=False),
    )
    return k(cols_flat, vals_flat, rowpat)


def _mm_body(m_ref, h_ref, o_ref):
    m = m_ref[0]
    h1 = jnp.dot(m, h_ref[0], preferred_element_type=jnp.float32)
    o_ref[0] = jnp.dot(m, h1, preferred_element_type=jnp.float32)


def _two_layer_mm(m, h):
    return pl.pallas_call(
        _mm_body,
        grid=(B,),
        in_specs=[
            pl.BlockSpec((1, N, N), lambda b: (b, 0, 0)),
            pl.BlockSpec((1, N, D), lambda b: (b, 0, 0)),
        ],
        out_specs=pl.BlockSpec((1, N, D), lambda b: (b, 0, 0)),
        out_shape=jax.ShapeDtypeStruct((B, N, D), jnp.float32),
    )(m, h)


@jax.jit
def kernel(node_hidden, in_node_index, in_node_mask):
    idx4 = in_node_index.reshape(NCHUNK, C // LANES, LANES, K)
    cols_flat = idx4.transpose(0, 1, 3, 2).reshape(-1)
    msk4 = in_node_mask.reshape(NCHUNK, C // LANES, LANES, K)
    vals_flat = msk4.transpose(0, 1, 3, 2).reshape(-1)

    g = np.arange(PAIRS)
    rowpat = jnp.asarray(((g // (K * LANES)) * LANES + g % LANES) * N,
                         dtype=jnp.int32)

    m = _build_m_sc(cols_flat, vals_flat, rowpat)
    m = m.reshape(B, N, N)
    return _two_layer_mm(m, node_hidden)


# speedup vs baseline: 100.9771x; 1.2797x over previous
import functools
import numpy as np
import jax
import jax.numpy as jnp
from jax import lax
from jax.experimental import pallas as pl
from jax.experimental.pallas import tpu as pltpu
from jax.experimental.pallas import tpu_sc as plsc

B, N, K, D = 16, 512, 32, 128
C = 128
NCHUNK = (B * N) // C
NW = 32
CHUNKS_PER_W = NCHUNK // NW
PAIRS = C * K
LANES = 16
GROUPS = PAIRS // LANES
RBLK = C // LANES


def _sc_scatter_body(cols_hbm, vals_hbm, m_hbm,
                     idx_v, val_v, acc_v):
    wid = lax.axis_index("s") * 2 + lax.axis_index("c")
    for cc in range(CHUNKS_PER_W):
        chunk = wid * CHUNKS_PER_W + cc
        base = chunk * PAIRS
        pltpu.sync_copy(cols_hbm.at[pl.ds(base, PAIRS)], idx_v)
        pltpu.sync_copy(vals_hbm.at[pl.ds(base, PAIRS)], val_v)

        zeros = jnp.zeros((LANES,), jnp.float32)

        def zero_blk(i, carry):
            base0 = i * (LANES * 16)
            for j in range(16):
                acc_v[pl.ds(base0 + j * LANES, LANES)] = zeros
            return carry

        lax.fori_loop(0, (C * N) // (LANES * 16), zero_blk, 0)

        def group(g, carry):
            off = g * LANES
            cols = idx_v[pl.ds(off, LANES)]
            vals = val_v[pl.ds(off, LANES)]
            rowbase = (g // (K // LANES)) * N
            plsc.addupdate_scatter(acc_v, [rowbase + cols], vals)
            return carry

        lax.fori_loop(0, GROUPS, group, 0)
        pltpu.sync_copy(acc_v, m_hbm.at[chunk])


def _build_m_sc(cols_flat, vals_flat):
    mesh = plsc.VectorSubcoreMesh(core_axis_name="c", subcore_axis_name="s",
                                  num_cores=2, num_subcores=16)
    k = pl.kernel(
        _sc_scatter_body,
        out_type=jax.ShapeDtypeStruct((NCHUNK, C * N), jnp.float32),
        mesh=mesh,
        scratch_types=[
            pltpu.VMEM((PAIRS,), jnp.int32),
            pltpu.VMEM((PAIRS,), jnp.float32),
            pltpu.VMEM((C * N,), jnp.float32),
        ],
        compiler_params=pltpu.CompilerParams(
            needs_layout_passes=False, use_tc_tiling_on_sc=False),
    )
    return k(cols_flat, vals_flat)


def _mm_body(m_ref, h_ref, o_ref):
    m = m_ref[0]
    h1 = jnp.dot(m, h_ref[0], preferred_element_type=jnp.float32)
    o_ref[0] = jnp.dot(m, h1, preferred_element_type=jnp.float32)


def _two_layer_mm(m, h):
    return pl.pallas_call(
        _mm_body,
        grid=(B,),
        in_specs=[
            pl.BlockSpec((1, N, N), lambda b: (b, 0, 0)),
            pl.BlockSpec((1, N, D), lambda b: (b, 0, 0)),
        ],
        out_specs=pl.BlockSpec((1, N, D), lambda b: (b, 0, 0)),
        out_shape=jax.ShapeDtypeStruct((B, N, D), jnp.float32),
    )(m, h)


@jax.jit
def kernel(node_hidden, in_node_index, in_node_mask):
    cols_flat = in_node_index.reshape(-1)
    vals_flat = in_node_mask.reshape(-1)

    m = _build_m_sc(cols_flat, vals_flat)
    m = m.reshape(B, N, N)
    return _two_layer_mm(m, node_hidden)
